# baseline (speedup 1.0000x reference)
import functools

import jax
import jax.numpy as jnp
from jax import lax
from jax.experimental import pallas as pl
from jax.experimental.pallas import tpu as pltpu
from jax.experimental.pallas import tpu_sc as plsc

VOCAB = 1000
D_MODEL = 128
BIGRAM_VOCAB = 1000000
BIGRAM_DIM = 64
B, S = 1024, 20
N = B * S
MOD = BIGRAM_VOCAB - 1

CHUNK = 128
LANES = 16
PAD = 8


def _sc_gather(tokens_flat, tok_emb, big_emb):
    info = plsc.get_sparse_core_info()
    nc, ns = info.num_cores, info.num_subcores
    nw = nc * ns
    per_w = N // nw
    nch = per_w // CHUNK
    n_fired = per_w - per_w // S

    mesh = plsc.VectorSubcoreMesh(core_axis_name="c", subcore_axis_name="s")

    @functools.partial(
        pl.kernel,
        mesh=mesh,
        out_type=[
            jax.ShapeDtypeStruct((N, D_MODEL), jnp.float32),
            jax.ShapeDtypeStruct((N, BIGRAM_DIM), jnp.float32),
        ],
        scratch_types=[
            pltpu.VMEM((PAD + per_w,), jnp.int32),
            pltpu.VMEM((per_w + LANES,), jnp.int32),
            pltpu.VMEM((2, CHUNK, D_MODEL), jnp.float32),
            pltpu.VMEM((per_w, BIGRAM_DIM), jnp.float32),
            pltpu.SemaphoreType.DMA,
            pltpu.SemaphoreType.DMA,
        ],
    )
    def k(tok_hbm, temb_hbm, bemb_hbm, xtok_hbm, hb_hbm,
          tokv, bidx, trows, brows, tsem, bsem):
        wid = lax.axis_index("s") * nc + lax.axis_index("c")
        base = wid * per_w

        pltpu.sync_copy(tok_hbm.at[pl.ds(base, per_w)],
                        tokv.at[pl.ds(PAD, per_w)])

        lane = lax.broadcasted_iota(jnp.int32, (LANES,), 0)
        for v in range(per_w // LANES):
            j0 = v * LANES
            curr = tokv[pl.ds(PAD + j0, LANES)]
            prev = tokv[pl.ds(PAD + j0 - 1, LANES)]
            h = lax.rem(lax.bitwise_xor(curr * 36313, prev * 27191),
                        jnp.int32(MOD))
            s = lax.rem(lane + jnp.int32(j0 % S), jnp.int32(S))
            bidx[pl.ds(j0, LANES)] = jnp.where(s == 0, jnp.int32(MOD), h)

        def fire(j, carry):
            idx = bidx[pl.ds(j, LANES)][0]

            @pl.when(lax.rem(j, S) != 0)
            def _():
                pltpu.make_async_copy(bemb_hbm.at[pl.ds(idx, 1)],
                                      brows.at[pl.ds(j, 1)], bsem).start()
            return carry

        lax.fori_loop(0, per_w, fire, 0)

        cps = []
        for c in range(nch):
            t_cp = pltpu.make_async_copy(
                temb_hbm.at[tokv.at[pl.ds(PAD + c * CHUNK, CHUNK)]],
                trows.at[c % 2], tsem)
            t_cp.start()
            cps.append(t_cp)
            if c > 0:
                cps[c - 1].wait()
                out_sl = pl.ds(base + (c - 1) * CHUNK, CHUNK)
                pltpu.sync_copy(trows.at[(c - 1) % 2], xtok_hbm.at[out_sl])
        cps[nch - 1].wait()
        out_sl = pl.ds(base + (nch - 1) * CHUNK, CHUNK)
        pltpu.sync_copy(trows.at[(nch - 1) % 2], xtok_hbm.at[out_sl])

        pltpu.make_async_copy(bemb_hbm.at[pl.ds(0, n_fired)],
                              brows.at[pl.ds(0, n_fired)], bsem).wait()
        pltpu.sync_copy(brows, hb_hbm.at[pl.ds(base, per_w)])

    return k(tokens_flat, tok_emb, big_emb)


BATCH_BLK = 64
BLK = BATCH_BLK * S


def _tc_body(xtok_ref, hb_ref, hb0_ref, emb_ref, pw_ref, bs_ref, g_ref,
             ns_ref, out_ref):
    row = lax.broadcasted_iota(jnp.int32, (BLK, 1), 0)
    s0 = lax.rem(row, S) == 0
    hb = jnp.where(s0, hb0_ref[...], hb_ref[...])
    hbp = lax.dot_general(hb, pw_ref[...],
                          (((1,), (1,)), ((), ())),
                          preferred_element_type=jnp.float32)
    x = xtok_ref[...] + hbp * bs_ref[0, 0]
    g = jax.nn.sigmoid(g_ref[...])
    xs = jnp.concatenate(
        [jnp.zeros((1, D_MODEL), jnp.float32), x[:-1, :]], axis=0)
    xprev = jnp.where(s0, 0.0, xs)
    x = (1.0 - g) * x + g * xprev
    ms = jnp.mean(x * x, axis=1, keepdims=True)
    xn = x * lax.rsqrt(ms + 1e-6) * ns_ref[...]
    logits = lax.dot_general(xn, emb_ref[...],
                             (((1,), (1,)), ((), ())),
                             preferred_element_type=jnp.float32)
    out_ref[...] = logits.reshape(BATCH_BLK, S, VOCAB)


def _tc_dense(x_tok, hb, hb0, tok_emb, proj_w, big_scale, gate, norm_scale):
    grid = (B // BATCH_BLK,)
    return pl.pallas_call(
        _tc_body,
        grid=grid,
        in_specs=[
            pl.BlockSpec((BLK, D_MODEL), lambda i: (i, 0)),
            pl.BlockSpec((BLK, BIGRAM_DIM), lambda i: (i, 0)),
            pl.BlockSpec((1, BIGRAM_DIM), lambda i: (0, 0)),
            pl.BlockSpec((VOCAB, D_MODEL), lambda i: (0, 0)),
            pl.BlockSpec((D_MODEL, BIGRAM_DIM), lambda i: (0, 0)),
            pl.BlockSpec((1, 1), lambda i: (0, 0)),
            pl.BlockSpec((1, D_MODEL), lambda i: (0, 0)),
            pl.BlockSpec((1, D_MODEL), lambda i: (0, 0)),
        ],
        out_specs=pl.BlockSpec((BATCH_BLK, S, VOCAB), lambda i: (i, 0, 0)),
        out_shape=jax.ShapeDtypeStruct((B, S, VOCAB), jnp.float32),
    )(x_tok, hb, hb0, tok_emb, proj_w, big_scale, gate, norm_scale)


def kernel(tokens, tok_emb, big_emb, proj_w, big_scale, gate, norm_scale):
    tokens_flat = tokens.reshape(-1).astype(jnp.int32)
    x_tok, hb = _sc_gather(tokens_flat, tok_emb, big_emb)
    hb0 = lax.slice(big_emb, (MOD, 0), (MOD + 1, BIGRAM_DIM))
    return _tc_dense(x_tok, hb, hb0, tok_emb, proj_w,
                     big_scale.reshape(1, 1).astype(jnp.float32),
                     gate.reshape(1, D_MODEL),
                     norm_scale.reshape(1, D_MODEL))
